# baseline (speedup 1.0000x reference)
import jax
import jax.numpy as jnp
from jax import lax
from jax.experimental import pallas as pl
from jax.experimental.pallas import tpu as pltpu
from jax.experimental.pallas import tpu_sc as plsc

VOCAB = 1_000_000
D = 64
T = 200
B = 4096
FLAT = B * T
NC = 2
NS = 16
NW = NC * NS
PER_W = FLAT // NW
CHUNK = 128
NCHUNK = PER_W // CHUNK
NBUF = 4
NGRP = NCHUNK // NBUF
GROUP = NBUF * CHUNK
LANES = 16
KD = D // LANES


def _body(idx_hbm, tok_hbm, pos_hbm, out_hbm,
          ix0, ix1, tt0, tt1, rows_v, ob0, ob1,
          sgi0, sgi1, sg0, sg1, sg2, sg3, sp0, sp1, sp2, sp3, so0, so1):
    cid = lax.axis_index("c")
    sid = lax.axis_index("s")
    wid = sid * NC + cid
    w0 = wid * PER_W
    ixg = [ix0, ix1]
    ttg = [tt0, tt1]
    sgi = [sgi0, sgi1]
    sg = [sg0, sg1, sg2, sg3]
    sp = [sp0, sp1, sp2, sp3]
    so = [so0, so1]
    outb = [ob0, ob1]

    def idx_start(g, p):
        pltpu.async_copy(idx_hbm.at[pl.ds(w0 + g * GROUP, GROUP)], ixg[p], sgi[p])

    def idx_wait(g, p):
        pltpu.make_async_copy(
            idx_hbm.at[pl.ds(w0 + g * GROUP, GROUP)], ixg[p], sgi[p]).wait()

    def build_tt(g, p):
        start = lax.rem(w0 + g * GROUP, T)
        for v in range(GROUP // LANES):
            sl = pl.ds(v * LANES, LANES)
            ttg[p][sl] = lax.rem(start + v * LANES + lax.iota(jnp.int32, LANES), T)

    def gather_start(b, p):
        pltpu.async_copy(tok_hbm.at[ixg[p].at[pl.ds(b * CHUNK, CHUNK)]],
                         rows_v.at[b], sg[b])

    def gather_wait(b):
        pltpu.make_async_copy(tok_hbm.at[pl.ds(0, CHUNK)], rows_v.at[b], sg[b]).wait()

    def pos_add_start(b, p):
        pltpu.async_copy(pos_hbm.at[ttg[p].at[pl.ds(b * CHUNK, CHUNK)]],
                         rows_v.at[b], sp[b], add=True)

    def pos_add_wait(b):
        pltpu.make_async_copy(pos_hbm.at[pl.ds(0, CHUNK)], rows_v.at[b], sp[b]).wait()

    def out_start(ci, ob):
        pltpu.async_copy(outb[ob], out_hbm.at[pl.ds(w0 + ci * CHUNK, CHUNK)], so[ob])

    def out_wait(ci, ob):
        pltpu.make_async_copy(
            outb[ob], out_hbm.at[pl.ds(w0 + ci * CHUNK, CHUNK)], so[ob]).wait()

    def extract(b, ob):
        @plsc.parallel_loop(0, CHUNK, step=1, unroll=4)
        def _(r):
            for k in range(KD):
                sl = pl.ds(k * LANES, LANES)
                outb[ob][r, sl] = rows_v[b, r, sl]

    idx_start(0, 0)
    idx_wait(0, 0)
    build_tt(0, 0)
    for b in range(NBUF):
        gather_start(b, 0)

    def phase(g, pv):
        pn = (pv + 1) % 2

        @pl.when(g < NGRP - 1)
        def _():
            idx_start(g + 1, pn)

        for b in range(NBUF):
            gather_wait(b)
            pos_add_start(b, pv)

        @pl.when(g < NGRP - 1)
        def _():
            idx_wait(g + 1, pn)
            build_tt(g + 1, pn)

        for b in range(NBUF):
            ci = NBUF * g + b
            ob = b % 2
            pos_add_wait(b)
            if b < 2:
                @pl.when(g > 0)
                def _():
                    out_wait(ci - 2, ob)
            else:
                out_wait(ci - 2, ob)
            extract(b, ob)
            out_start(ci, ob)

            @pl.when(g < NGRP - 1)
            def _():
                gather_start(b, pn)

    def g_body(go, _):
        phase(2 * go, 0)
        phase(2 * go + 1, 1)
        return ()

    lax.fori_loop(0, NGRP // 2, g_body, ())
    out_wait(NCHUNK - 2, 0)
    out_wait(NCHUNK - 1, 1)


@jax.jit
def _embed(idx1d, tokp, posp):
    mesh = plsc.VectorSubcoreMesh(core_axis_name="c", subcore_axis_name="s")
    f = pl.kernel(
        _body,
        mesh=mesh,
        out_type=jax.ShapeDtypeStruct((FLAT, D), jnp.float32),
        scratch_types=[
            pltpu.VMEM((GROUP,), jnp.int32),
            pltpu.VMEM((GROUP,), jnp.int32),
            pltpu.VMEM((GROUP,), jnp.int32),
            pltpu.VMEM((GROUP,), jnp.int32),
            pltpu.VMEM((NBUF, CHUNK, 2 * D), jnp.float32),
            pltpu.VMEM((CHUNK, D), jnp.float32),
            pltpu.VMEM((CHUNK, D), jnp.float32),
        ] + [pltpu.SemaphoreType.DMA] * 12,
        compiler_params=pltpu.CompilerParams(use_tc_tiling_on_sc=True),
    )
    return f(idx1d, tokp, posp)


def kernel(idx, token_embedding_table, position_embedding_table):
    idx1d = idx.astype(jnp.int32).reshape(FLAT)
    tokp = jnp.pad(token_embedding_table, ((0, 0), (0, D)))
    posp = jnp.pad(position_embedding_table, ((0, 0), (0, D)))
    out = _embed(idx1d, tokp, posp)
    return out.reshape(B, T, D)
